# baseline (speedup 1.0000x reference)
import functools

import jax
import jax.numpy as jnp
from jax.experimental import pallas as pl
from jax.experimental.pallas import tpu as pltpu

LO = 1e-5
SKIP = -5.0
LOG2E = 1.4426950408889634
LN2 = 0.6931471805599453
PRESCALE = 7.0
C_STEP = 2.0 ** (SKIP * LOG2E + PRESCALE)


def _ctc_kernel(in_ref, cls_ref, out_ref, lm_ref, a_ref, aux_ref, *,
                s_chunk, t_pad, n_sc, n_valid_t, s_total):
    sc = pl.program_id(1)
    G = in_ref.shape[1]
    C = in_ref.shape[2]

    x = in_ref[...]
    xc = jnp.maximum(x, LO)
    ssum = jnp.sum(xc, axis=2, keepdims=True)
    p = (xc * ((2.0 ** PRESCALE) / ssum)).astype(jnp.bfloat16)
    iota_c = jax.lax.broadcasted_iota(jnp.int32, (C, t_pad), 0)
    for n in range(G):
        cls_row = cls_ref[n, :].reshape(1, t_pad)
        oh = jnp.where(iota_c == cls_row, 1.0, 0.0).astype(jnp.bfloat16)
        pn = p[:, n, :].reshape(s_chunk, C)
        lm_ref[:, n, :] = jnp.dot(
            pn, oh, preferred_element_type=jnp.float32).astype(jnp.bfloat16)

    @pl.when(sc == 0)
    def _():
        lane = jax.lax.broadcasted_iota(jnp.int32, (G, t_pad), 1)
        t_of_lane = 4 * (lane & 127) + (lane >> 7)
        a_ref[...] = jnp.exp2((SKIP * LOG2E) * t_of_lane.astype(jnp.float32))
        aux_ref[:, 0:128] = jnp.ones((G, 128), jnp.float32)
        aux_ref[:, 128:256] = jnp.zeros((G, 128), jnp.float32)

    lane_i = jax.lax.broadcasted_iota(jnp.int32, (G, 128), 1)
    UNROLL = 16

    def body(i, carry):
        a0, a1, a2, a3, inj, led = carry
        mx = None
        for u in range(UNROLL):
            m = lm_ref[i * UNROLL + u].astype(jnp.float32)
            w0 = pltpu.roll(a3, 1, axis=1)
            w0 = jnp.where(lane_i == 0, inj, w0)
            n0 = (a0 + w0) * m[:, 0:128]
            n1 = (a1 + a0) * m[:, 128:256]
            n2 = (a2 + a1) * m[:, 256:384]
            n3 = (a3 + a2) * m[:, 384:512]
            inj = inj * C_STEP
            a0, a1, a2, a3 = n0, n1, n2, n3
            if u == 8:
                mm = jnp.maximum(jnp.maximum(a0, a1), jnp.maximum(a2, a3))
                mx = jnp.max(mm, axis=1, keepdims=True)
        mx = jnp.maximum(mx, 1e-30)
        r = 1.0 / mx
        a0, a1, a2, a3 = a0 * r, a1 * r, a2 * r, a3 * r
        inj = inj * r
        led = led + jnp.log2(mx)
        return a0, a1, a2, a3, inj, led

    carry0 = (a_ref[:, 0:128], a_ref[:, 128:256],
              a_ref[:, 256:384], a_ref[:, 384:512],
              aux_ref[:, 0:128], aux_ref[:, 128:256])
    a0, a1, a2, a3, inj, led = jax.lax.fori_loop(
        0, s_chunk // UNROLL, body, carry0)
    a_ref[:, 0:128] = a0
    a_ref[:, 128:256] = a1
    a_ref[:, 256:384] = a2
    a_ref[:, 384:512] = a3
    aux_ref[:, 0:128] = inj
    aux_ref[:, 128:256] = led

    @pl.when(sc == n_sc - 1)
    def _():
        vs = (a0, a1, a2, a3)
        t1, t2 = n_valid_t - 1, n_valid_t - 2
        x1 = vs[t1 % 4][:, t1 // 4:t1 // 4 + 1]
        y1 = vs[t2 % 4][:, t2 // 4:t2 // 4 + 1]
        tot = jnp.log2(x1 + y1) + led[:, 0:1] - PRESCALE * s_total
        loss = (-LN2) * tot
        out_ref[...] = jnp.broadcast_to(loss, (G, 128)).reshape(1, G, 128)


@jax.jit
def kernel(input, targets):
    S, N, C = input.shape
    L = targets.shape[0]
    T = 2 * L + 1
    t_pad = 512
    G = N // 2
    s_chunk = 256
    n_sc = S // s_chunk

    tgt = targets.astype(jnp.int32)
    cls = jnp.zeros((N, t_pad), jnp.int32)
    cls = cls.at[:, 1:2 * L:2].set(tgt.T)
    cls = cls.reshape(N, t_pad // 4, 4).transpose(0, 2, 1).reshape(N, t_pad)

    out = pl.pallas_call(
        functools.partial(_ctc_kernel, s_chunk=s_chunk, t_pad=t_pad,
                          n_sc=n_sc, n_valid_t=T, s_total=S),
        grid=(2, n_sc),
        in_specs=[
            pl.BlockSpec((s_chunk, G, C), lambda g, sc: (sc, g, 0)),
            pl.BlockSpec((G, t_pad), lambda g, sc: (g, 0)),
        ],
        out_specs=pl.BlockSpec((1, G, 128), lambda g, sc: (g, 0, 0)),
        out_shape=jax.ShapeDtypeStruct((2, G, 128), jnp.float32),
        scratch_shapes=[
            pltpu.VMEM((s_chunk, G, t_pad), jnp.bfloat16),
            pltpu.VMEM((G, t_pad), jnp.float32),
            pltpu.VMEM((G, 256), jnp.float32),
        ],
        compiler_params=pltpu.CompilerParams(
            dimension_semantics=("parallel", "arbitrary"),
            vmem_limit_bytes=100 * 1024 * 1024,
        ),
    )(input, cls)

    losses = out.reshape(N, 128)[:, 0]
    return jnp.sum(losses) / N


# speedup vs baseline: 1.1922x; 1.1922x over previous
import functools

import jax
import jax.numpy as jnp
from jax.experimental import pallas as pl
from jax.experimental.pallas import tpu as pltpu

LO = 1e-5
SKIP = -5.0
LOG2E = 1.4426950408889634
LN2 = 0.6931471805599453
PRESCALE = 7.0
C_STEP = 2.0 ** (SKIP * LOG2E + PRESCALE)


def _ctc_kernel(in_ref, cls_ref, out_ref, lm_ref, a_ref, aux_ref, *,
                s_chunk, t_pad, n_sc, n_valid_t, s_total):
    sc = pl.program_id(1)
    G = in_ref.shape[1]
    C = in_ref.shape[2]

    x = in_ref[...]
    xc = jnp.maximum(x, LO)
    ssum = jnp.sum(xc, axis=2, keepdims=True)
    p = (xc * ((2.0 ** PRESCALE) / ssum)).astype(jnp.bfloat16)
    iota_c = jax.lax.broadcasted_iota(jnp.int32, (C, t_pad), 0)
    for n in range(G):
        cls_row = cls_ref[n, :].reshape(1, t_pad)
        oh = jnp.where(iota_c == cls_row, 1.0, 0.0).astype(jnp.bfloat16)
        pn = p[:, n, :].reshape(s_chunk, C)
        lm_ref[:, n, :] = jnp.dot(pn, oh, preferred_element_type=jnp.float32)

    @pl.when(sc == 0)
    def _():
        lane = jax.lax.broadcasted_iota(jnp.int32, (G, t_pad), 1)
        t_of_lane = 4 * (lane & 127) + (lane >> 7)
        a_ref[...] = jnp.exp2((SKIP * LOG2E) * t_of_lane.astype(jnp.float32))
        aux_ref[:, 0:128] = jnp.ones((G, 128), jnp.float32)
        aux_ref[:, 128:256] = jnp.zeros((G, 128), jnp.float32)

    lane_i = jax.lax.broadcasted_iota(jnp.int32, (G, 128), 1)
    UNROLL = 16

    def body(i, carry):
        a0, a1, a2, a3, inj, led = carry
        mx = None
        for u in range(UNROLL):
            m = lm_ref[i * UNROLL + u]
            w0 = pltpu.roll(a3, 1, axis=1)
            w0 = jnp.where(lane_i == 0, inj, w0)
            n0 = (a0 + w0) * m[:, 0:128]
            n1 = (a1 + a0) * m[:, 128:256]
            n2 = (a2 + a1) * m[:, 256:384]
            n3 = (a3 + a2) * m[:, 384:512]
            inj = inj * C_STEP
            a0, a1, a2, a3 = n0, n1, n2, n3
            if u == 8:
                mm = jnp.maximum(jnp.maximum(a0, a1), jnp.maximum(a2, a3))
                mx = jnp.max(mm, axis=1, keepdims=True)
        mx = jnp.maximum(mx, 1e-30)
        r = 1.0 / mx
        a0, a1, a2, a3 = a0 * r, a1 * r, a2 * r, a3 * r
        inj = inj * r
        led = led + jnp.log2(mx)
        return a0, a1, a2, a3, inj, led

    carry0 = (a_ref[:, 0:128], a_ref[:, 128:256],
              a_ref[:, 256:384], a_ref[:, 384:512],
              aux_ref[:, 0:128], aux_ref[:, 128:256])
    a0, a1, a2, a3, inj, led = jax.lax.fori_loop(
        0, s_chunk // UNROLL, body, carry0)
    a_ref[:, 0:128] = a0
    a_ref[:, 128:256] = a1
    a_ref[:, 256:384] = a2
    a_ref[:, 384:512] = a3
    aux_ref[:, 0:128] = inj
    aux_ref[:, 128:256] = led

    @pl.when(sc == n_sc - 1)
    def _():
        vs = (a0, a1, a2, a3)
        t1, t2 = n_valid_t - 1, n_valid_t - 2
        x1 = vs[t1 % 4][:, t1 // 4:t1 // 4 + 1]
        y1 = vs[t2 % 4][:, t2 // 4:t2 // 4 + 1]
        tot = jnp.log2(x1 + y1) + led[:, 0:1] - PRESCALE * s_total
        loss = (-LN2) * tot
        out_ref[...] = jnp.broadcast_to(loss, (G, 128)).reshape(1, G, 128)


@jax.jit
def kernel(input, targets):
    S, N, C = input.shape
    L = targets.shape[0]
    T = 2 * L + 1
    t_pad = 512
    G = N // 2
    s_chunk = 512
    n_sc = S // s_chunk

    tgt = targets.astype(jnp.int32)
    cls = jnp.zeros((N, t_pad), jnp.int32)
    cls = cls.at[:, 1:2 * L:2].set(tgt.T)
    cls = cls.reshape(N, t_pad // 4, 4).transpose(0, 2, 1).reshape(N, t_pad)

    out = pl.pallas_call(
        functools.partial(_ctc_kernel, s_chunk=s_chunk, t_pad=t_pad,
                          n_sc=n_sc, n_valid_t=T, s_total=S),
        grid=(2, n_sc),
        in_specs=[
            pl.BlockSpec((s_chunk, G, C), lambda g, sc: (sc, g, 0)),
            pl.BlockSpec((G, t_pad), lambda g, sc: (g, 0)),
        ],
        out_specs=pl.BlockSpec((1, G, 128), lambda g, sc: (g, 0, 0)),
        out_shape=jax.ShapeDtypeStruct((2, G, 128), jnp.float32),
        scratch_shapes=[
            pltpu.VMEM((s_chunk, G, t_pad), jnp.float32),
            pltpu.VMEM((G, t_pad), jnp.float32),
            pltpu.VMEM((G, 256), jnp.float32),
        ],
        compiler_params=pltpu.CompilerParams(
            dimension_semantics=("parallel", "arbitrary"),
            vmem_limit_bytes=100 * 1024 * 1024,
        ),
    )(input, cls)

    losses = out.reshape(N, 128)[:, 0]
    return jnp.sum(losses) / N


# speedup vs baseline: 1.2185x; 1.0220x over previous
import functools

import jax
import jax.numpy as jnp
from jax.experimental import pallas as pl
from jax.experimental.pallas import tpu as pltpu

LO = 1e-5
SKIP = -5.0
LOG2E = 1.4426950408889634
LN2 = 0.6931471805599453
PRESCALE = 7.0
C_STEP = 2.0 ** (SKIP * LOG2E + PRESCALE)


def _ctc_kernel(in_ref, cls_ref, out_ref, lm_ref, a_ref, aux_ref, *,
                s_chunk, t_pad, n_sc, n_valid_t, s_total):
    sc = pl.program_id(1)
    G = in_ref.shape[1]
    C = in_ref.shape[2]

    x = in_ref[...]
    xc = jnp.maximum(x, LO)
    ssum = jnp.sum(xc, axis=2, keepdims=True)
    p = (xc * ((2.0 ** PRESCALE) / ssum)).astype(jnp.bfloat16)
    iota_c = jax.lax.broadcasted_iota(jnp.int32, (C, t_pad), 0)
    for n in range(G):
        cls_row = cls_ref[n, :].reshape(1, t_pad)
        oh = jnp.where(iota_c == cls_row, 1.0, 0.0).astype(jnp.bfloat16)
        pn = p[:, n, :].reshape(s_chunk, C)
        lm_ref[:, n, :] = jnp.dot(pn, oh, preferred_element_type=jnp.float32)

    @pl.when(sc == 0)
    def _():
        lane = jax.lax.broadcasted_iota(jnp.int32, (G, t_pad), 1)
        t_of_lane = 4 * (lane & 127) + (lane >> 7)
        a_ref[...] = jnp.exp2((SKIP * LOG2E) * t_of_lane.astype(jnp.float32))
        aux_ref[:, 0:128] = jnp.ones((G, 128), jnp.float32)
        aux_ref[:, 128:256] = jnp.zeros((G, 128), jnp.float32)

    lane_i = jax.lax.broadcasted_iota(jnp.int32, (G, 128), 1)
    UNROLL = 32

    def body(i, carry):
        a0, a1, a2, a3, inj, led = carry
        mx = None
        for u in range(UNROLL):
            m = lm_ref[i * UNROLL + u]
            w0 = pltpu.roll(a3, 1, axis=1)
            w0 = jnp.where(lane_i == 0, inj, w0)
            n0 = (a0 + w0) * m[:, 0:128]
            n1 = (a1 + a0) * m[:, 128:256]
            n2 = (a2 + a1) * m[:, 256:384]
            n3 = (a3 + a2) * m[:, 384:512]
            inj = inj * C_STEP
            a0, a1, a2, a3 = n0, n1, n2, n3
            if u == 8:
                mm = jnp.maximum(jnp.maximum(a0, a1), jnp.maximum(a2, a3))
                mx = jnp.max(mm, axis=1, keepdims=True)
        mx = jnp.maximum(mx, 1e-30)
        r = 1.0 / mx
        a0, a1, a2, a3 = a0 * r, a1 * r, a2 * r, a3 * r
        inj = inj * r
        led = led + jnp.log2(mx)
        return a0, a1, a2, a3, inj, led

    carry0 = (a_ref[:, 0:128], a_ref[:, 128:256],
              a_ref[:, 256:384], a_ref[:, 384:512],
              aux_ref[:, 0:128], aux_ref[:, 128:256])
    a0, a1, a2, a3, inj, led = jax.lax.fori_loop(
        0, s_chunk // UNROLL, body, carry0)
    a_ref[:, 0:128] = a0
    a_ref[:, 128:256] = a1
    a_ref[:, 256:384] = a2
    a_ref[:, 384:512] = a3
    aux_ref[:, 0:128] = inj
    aux_ref[:, 128:256] = led

    @pl.when(sc == n_sc - 1)
    def _():
        vs = (a0, a1, a2, a3)
        t1, t2 = n_valid_t - 1, n_valid_t - 2
        x1 = vs[t1 % 4][:, t1 // 4:t1 // 4 + 1]
        y1 = vs[t2 % 4][:, t2 // 4:t2 // 4 + 1]
        tot = jnp.log2(x1 + y1) + led[:, 0:1] - PRESCALE * s_total
        loss = (-LN2) * tot
        out_ref[...] = jnp.broadcast_to(loss, (G, 128)).reshape(1, G, 128)


@jax.jit
def kernel(input, targets):
    S, N, C = input.shape
    L = targets.shape[0]
    T = 2 * L + 1
    t_pad = 512
    G = N // 2
    s_chunk = 512
    n_sc = S // s_chunk

    tgt = targets.astype(jnp.int32)
    cls = jnp.zeros((N, t_pad), jnp.int32)
    cls = cls.at[:, 1:2 * L:2].set(tgt.T)
    cls = cls.reshape(N, t_pad // 4, 4).transpose(0, 2, 1).reshape(N, t_pad)

    out = pl.pallas_call(
        functools.partial(_ctc_kernel, s_chunk=s_chunk, t_pad=t_pad,
                          n_sc=n_sc, n_valid_t=T, s_total=S),
        grid=(2, n_sc),
        in_specs=[
            pl.BlockSpec((s_chunk, G, C), lambda g, sc: (sc, g, 0)),
            pl.BlockSpec((G, t_pad), lambda g, sc: (g, 0)),
        ],
        out_specs=pl.BlockSpec((1, G, 128), lambda g, sc: (g, 0, 0)),
        out_shape=jax.ShapeDtypeStruct((2, G, 128), jnp.float32),
        scratch_shapes=[
            pltpu.VMEM((s_chunk, G, t_pad), jnp.float32),
            pltpu.VMEM((G, t_pad), jnp.float32),
            pltpu.VMEM((G, 256), jnp.float32),
        ],
        compiler_params=pltpu.CompilerParams(
            dimension_semantics=("parallel", "arbitrary"),
            vmem_limit_bytes=100 * 1024 * 1024,
        ),
    )(input, cls)

    losses = out.reshape(N, 128)[:, 0]
    return jnp.sum(losses) / N
